# baseline (speedup 1.0000x reference)
import functools

import jax
import jax.numpy as jnp
from jax import lax
from jax.experimental import pallas as pl
from jax.experimental.pallas import tpu as pltpu
from jax.experimental.pallas import tpu_sc as plsc

_NUM_CODES = 8192
_EMBED_DIM = 256
_BATCH = 36864
_COMMITMENT_COST = 0.25

_BM = 256
_NW = 32
_BPW = _BATCH // _NW
_GCH = 128
_LANES = 16


def _tree_min(x):
    w = x.shape[1]
    while w > 128:
        half = w // 2
        x = jnp.minimum(x[:, :half], x[:, half:])
        w = half
    return jnp.min(x, axis=1, keepdims=True)


_NG = _NUM_CODES // 32


def _argmin_body(z_ref, sumz_ref, et_ref, idx_ref, losssum_ref, counts_ref,
                 acc_ref, sume_ref, g_ref, cacc_ref):
    i = pl.program_id(0)
    n = et_ref.shape[1]
    bm = z_ref.shape[0]

    @pl.when(i == 0)
    def _init():
        et = et_ref[...]
        sume_ref[...] = jnp.sum(et * et, axis=0, keepdims=True)
        jj = lax.broadcasted_iota(jnp.int32, (n, _NG), 0)
        gg = lax.broadcasted_iota(jnp.int32, (n, _NG), 1)
        w = lax.bitcast_convert_type((127 - 2 * (jj & 31)) << 23, jnp.float32)
        g_ref[...] = jnp.where((jj >> 5) == gg, w, jnp.float32(0.0))
        acc_ref[0, 0] = 0.0
        cacc_ref[...] = jnp.zeros_like(cacc_ref)

    z = z_ref[...]
    mm = lax.dot_general(z, et_ref[...], (((1,), (0,)), ((), ())),
                         preferred_element_type=jnp.float32)
    d2 = (sumz_ref[...] + mm * -2.0) + sume_ref[...]
    dist = jnp.sqrt(jnp.maximum(d2, 0.0))
    vmin = _tree_min(dist)
    maskf = (dist == vmin).astype(jnp.float32)
    pres = lax.dot_general(maskf, g_ref[...], (((1,), (0,)), ((), ())),
                           preferred_element_type=jnp.float32)
    gidf = lax.broadcasted_iota(jnp.int32, (1, _NG), 1).astype(jnp.float32)
    firstg = _tree_min(jnp.where(pres > 0.0, gidf, jnp.float32(jnp.inf)))
    ohg = (gidf == firstg).astype(jnp.float32)
    psel = jnp.sum(pres * ohg, axis=1, keepdims=True)
    eb = (lax.bitcast_convert_type(psel, jnp.int32) >> 23) & 255
    kk = (127 - eb) >> 1
    first_i = firstg.astype(jnp.int32) * 32 + kk
    idx_ref[...] = first_i[:, 0]
    ohk = (lax.broadcasted_iota(jnp.int32, (bm, 32), 1)
           == kk).astype(jnp.float32)
    cacc_ref[...] += lax.dot_general(ohg, ohk, (((0,), (0,)), ((), ())),
                                     preferred_element_type=jnp.float32)
    acc_ref[0, 0] += jnp.sum(vmin * vmin)

    @pl.when(i == pl.num_programs(0) - 1)
    def _done():
        losssum_ref[0, 0] = acc_ref[0, 0]
        counts_ref[...] = cacc_ref[...]


def _argmin_call(z_e, sumz, emb_t):
    m, k = z_e.shape
    n = emb_t.shape[1]
    return pl.pallas_call(
        _argmin_body,
        grid=(m // _BM,),
        in_specs=[
            pl.BlockSpec((_BM, k), lambda i: (i, 0)),
            pl.BlockSpec((_BM, 1), lambda i: (i, 0)),
            pl.BlockSpec((k, n), lambda i: (0, 0)),
        ],
        out_specs=[
            pl.BlockSpec((_BM,), lambda i: (i,)),
            pl.BlockSpec(memory_space=pltpu.SMEM),
            pl.BlockSpec((_NG, 32), lambda i: (0, 0)),
        ],
        out_shape=[
            jax.ShapeDtypeStruct((m,), jnp.int32),
            jax.ShapeDtypeStruct((1, 1), jnp.float32),
            jax.ShapeDtypeStruct((_NG, 32), jnp.float32),
        ],
        scratch_shapes=[
            pltpu.SMEM((1, 1), jnp.float32),
            pltpu.VMEM((1, n), jnp.float32),
            pltpu.VMEM((n, _NG), jnp.float32),
            pltpu.VMEM((_NG, 32), jnp.float32),
        ],
        compiler_params=pltpu.CompilerParams(
            dimension_semantics=("arbitrary",)),
    )(z_e, sumz, emb_t)


_NCH = _BPW // _GCH
_STRIPE = _NUM_CODES // 16


def _sc_gather_body(emb_hbm, idx_hbm, zq_hbm, idx_v, rows_v, sem):
    wid = lax.axis_index("s") * 2 + lax.axis_index("c")
    base = wid * _BPW

    def _ldidx(j, c):
        pltpu.sync_copy(idx_hbm.at[pl.ds(base + j * _GCH, _GCH)], idx_v.at[j])
        return c
    lax.fori_loop(0, _NCH, _ldidx, 0)

    def _gather(j, c):
        pltpu.async_copy(emb_hbm.at[idx_v.at[j]], rows_v, sem).wait()
        pltpu.sync_copy(rows_v, zq_hbm.at[pl.ds(base + j * _GCH, _GCH)])
        return c
    lax.fori_loop(0, _NCH, _gather, 0)


def _sc_gather(embedding, indices):
    mesh = plsc.VectorSubcoreMesh(core_axis_name="c", subcore_axis_name="s")
    fn = functools.partial(
        pl.kernel,
        mesh=mesh,
        out_type=jax.ShapeDtypeStruct((_BATCH, _EMBED_DIM), jnp.float32),
        scratch_types=[
            pltpu.VMEM((_NCH, _GCH), jnp.int32),
            pltpu.VMEM((_GCH, _EMBED_DIM), jnp.float32),
            pltpu.SemaphoreType.DMA,
        ],
    )(_sc_gather_body)
    return fn(embedding, indices)


def _perp_body(pc_ref, out_ref):
    counts = pc_ref[...]
    p = counts / jnp.float32(_BATCH)
    ent = jnp.sum(p * jnp.log(p + 1e-10))
    out_ref[0, 0] = jnp.exp(-ent)


def _perp_call(pcounts):
    return pl.pallas_call(
        _perp_body,
        out_specs=pl.BlockSpec(memory_space=pltpu.SMEM),
        out_shape=jax.ShapeDtypeStruct((1, 1), jnp.float32),
    )(pcounts)


def kernel(z_e, embedding):
    emb_t = embedding.T
    sumz = jnp.sum(z_e * z_e, axis=1, keepdims=True)
    indices, loss_sum, counts2d = _argmin_call(z_e, sumz, emb_t)
    z_q = _sc_gather(embedding, indices)
    perp = _perp_call(counts2d.reshape(1, _NUM_CODES))
    codebook_loss = loss_sum[0, 0] / jnp.float32(_BATCH * _EMBED_DIM)
    commitment_loss = _COMMITMENT_COST * codebook_loss
    return (z_q, indices, codebook_loss, commitment_loss, perp[0, 0])


# speedup vs baseline: 1.0111x; 1.0111x over previous
import functools

import jax
import jax.numpy as jnp
from jax import lax
from jax.experimental import pallas as pl
from jax.experimental.pallas import tpu as pltpu
from jax.experimental.pallas import tpu_sc as plsc

_NUM_CODES = 8192
_EMBED_DIM = 256
_BATCH = 36864
_COMMITMENT_COST = 0.25

_BM = 256
_NW = 32
_BPW = _BATCH // _NW
_GCH = 128
_LANES = 16


def _tree_min(x):
    w = x.shape[1]
    while w > 128:
        half = w // 2
        x = jnp.minimum(x[:, :half], x[:, half:])
        w = half
    return jnp.min(x, axis=1, keepdims=True)


_NG = _NUM_CODES // 32


def _argmin_body(z_ref, sumz_ref, et_ref, idx_ref, losssum_ref, counts_ref,
                 acc_ref, sume_ref, g_ref, gid_ref, cacc_ref):
    i = pl.program_id(0)
    n = et_ref.shape[1]
    bm = z_ref.shape[0]

    @pl.when(i == 0)
    def _init():
        et = et_ref[...]
        sume_ref[...] = jnp.sum(et * et, axis=0, keepdims=True)
        jj = lax.broadcasted_iota(jnp.int32, (n, _NG), 0)
        gg = lax.broadcasted_iota(jnp.int32, (n, _NG), 1)
        w = lax.bitcast_convert_type((127 - 2 * (jj & 31)) << 23, jnp.float32)
        g_ref[...] = jnp.where((jj >> 5) == gg, w,
                               jnp.float32(0.0)).astype(jnp.bfloat16)
        gid_ref[...] = lax.broadcasted_iota(
            jnp.int32, (1, _NG), 1).astype(jnp.float32)
        acc_ref[0, 0] = 0.0
        cacc_ref[...] = jnp.zeros_like(cacc_ref)

    z = z_ref[...]
    mm = lax.dot_general(z, et_ref[...], (((1,), (0,)), ((), ())),
                         preferred_element_type=jnp.float32)
    d2 = (sumz_ref[...] + mm * -2.0) + sume_ref[...]
    dist = jnp.sqrt(jnp.maximum(d2, 0.0))
    vmin = _tree_min(dist)
    maskb = (dist == vmin).astype(jnp.bfloat16)
    pres = lax.dot_general(maskb, g_ref[...], (((1,), (0,)), ((), ())),
                           preferred_element_type=jnp.float32)
    gidf = gid_ref[...]
    firstg = _tree_min(jnp.where(pres > 0.0, gidf, jnp.float32(jnp.inf)))
    ohg = (gidf == firstg).astype(jnp.float32)
    psel = jnp.sum(pres * ohg, axis=1, keepdims=True)
    eb = (lax.bitcast_convert_type(psel, jnp.int32) >> 23) & 255
    kk = (127 - eb) >> 1
    first_i = firstg.astype(jnp.int32) * 32 + kk
    idx_ref[...] = first_i[:, 0]
    ohk = (lax.broadcasted_iota(jnp.int32, (bm, 32), 1)
           == kk).astype(jnp.bfloat16)
    cacc_ref[...] += lax.dot_general(ohg.astype(jnp.bfloat16), ohk,
                                     (((0,), (0,)), ((), ())),
                                     preferred_element_type=jnp.float32)
    acc_ref[0, 0] += jnp.sum(vmin * vmin)

    @pl.when(i == pl.num_programs(0) - 1)
    def _done():
        losssum_ref[0, 0] = acc_ref[0, 0]
        counts_ref[...] = cacc_ref[...]


def _argmin_call(z_e, sumz, emb_t):
    m, k = z_e.shape
    n = emb_t.shape[1]
    return pl.pallas_call(
        _argmin_body,
        grid=(m // _BM,),
        in_specs=[
            pl.BlockSpec((_BM, k), lambda i: (i, 0)),
            pl.BlockSpec((_BM, 1), lambda i: (i, 0)),
            pl.BlockSpec((k, n), lambda i: (0, 0)),
        ],
        out_specs=[
            pl.BlockSpec((_BM,), lambda i: (i,)),
            pl.BlockSpec(memory_space=pltpu.SMEM),
            pl.BlockSpec((_NG, 32), lambda i: (0, 0)),
        ],
        out_shape=[
            jax.ShapeDtypeStruct((m,), jnp.int32),
            jax.ShapeDtypeStruct((1, 1), jnp.float32),
            jax.ShapeDtypeStruct((_NG, 32), jnp.float32),
        ],
        scratch_shapes=[
            pltpu.SMEM((1, 1), jnp.float32),
            pltpu.VMEM((1, n), jnp.float32),
            pltpu.VMEM((n, _NG), jnp.bfloat16),
            pltpu.VMEM((1, _NG), jnp.float32),
            pltpu.VMEM((_NG, 32), jnp.float32),
        ],
        compiler_params=pltpu.CompilerParams(
            dimension_semantics=("arbitrary",)),
    )(z_e, sumz, emb_t)


_NCH = _BPW // _GCH
_STRIPE = _NUM_CODES // 16


def _sc_gather_body(emb_hbm, idx_hbm, zq_hbm, idx_v, rows_v, sem):
    wid = lax.axis_index("s") * 2 + lax.axis_index("c")
    base = wid * _BPW

    def _ldidx(j, c):
        pltpu.sync_copy(idx_hbm.at[pl.ds(base + j * _GCH, _GCH)], idx_v.at[j])
        return c
    lax.fori_loop(0, _NCH, _ldidx, 0)

    def _gather(j, c):
        pltpu.async_copy(emb_hbm.at[idx_v.at[j]], rows_v, sem).wait()
        pltpu.sync_copy(rows_v, zq_hbm.at[pl.ds(base + j * _GCH, _GCH)])
        return c
    lax.fori_loop(0, _NCH, _gather, 0)


def _sc_gather(embedding, indices):
    mesh = plsc.VectorSubcoreMesh(core_axis_name="c", subcore_axis_name="s")
    fn = functools.partial(
        pl.kernel,
        mesh=mesh,
        out_type=jax.ShapeDtypeStruct((_BATCH, _EMBED_DIM), jnp.float32),
        scratch_types=[
            pltpu.VMEM((_NCH, _GCH), jnp.int32),
            pltpu.VMEM((_GCH, _EMBED_DIM), jnp.float32),
            pltpu.SemaphoreType.DMA,
        ],
    )(_sc_gather_body)
    return fn(embedding, indices)


def _perp_body(pc_ref, out_ref):
    counts = pc_ref[...]
    p = counts / jnp.float32(_BATCH)
    ent = jnp.sum(p * jnp.log(p + 1e-10))
    out_ref[0, 0] = jnp.exp(-ent)


def _perp_call(pcounts):
    return pl.pallas_call(
        _perp_body,
        out_specs=pl.BlockSpec(memory_space=pltpu.SMEM),
        out_shape=jax.ShapeDtypeStruct((1, 1), jnp.float32),
    )(pcounts)


def kernel(z_e, embedding):
    emb_t = embedding.T
    sumz = jnp.sum(z_e * z_e, axis=1, keepdims=True)
    indices, loss_sum, counts2d = _argmin_call(z_e, sumz, emb_t)
    z_q = _sc_gather(embedding, indices)
    perp = _perp_call(counts2d.reshape(1, _NUM_CODES))
    codebook_loss = loss_sum[0, 0] / jnp.float32(_BATCH * _EMBED_DIM)
    commitment_loss = _COMMITMENT_COST * codebook_loss
    return (z_q, indices, codebook_loss, commitment_loss, perp[0, 0])


# speedup vs baseline: 1.0656x; 1.0538x over previous
import functools

import jax
import jax.numpy as jnp
from jax import lax
from jax.experimental import pallas as pl
from jax.experimental.pallas import tpu as pltpu
from jax.experimental.pallas import tpu_sc as plsc

_NUM_CODES = 8192
_EMBED_DIM = 256
_BATCH = 36864
_COMMITMENT_COST = 0.25

_BM = 512
_NW = 32
_BPW = _BATCH // _NW
_GCH = 128
_LANES = 16


def _tree_min(x):
    w = x.shape[1]
    while w > 128:
        half = w // 2
        x = jnp.minimum(x[:, :half], x[:, half:])
        w = half
    return jnp.min(x, axis=1, keepdims=True)


_NG = _NUM_CODES // 32


def _argmin_body(z_ref, sumz_ref, et_ref, idx_ref, losssum_ref, counts_ref,
                 acc_ref, sume_ref, g_ref, gid_ref, cacc_ref):
    i = pl.program_id(0)
    n = et_ref.shape[1]
    bm = z_ref.shape[0]

    @pl.when(i == 0)
    def _init():
        et = et_ref[...]
        sume_ref[...] = jnp.sum(et * et, axis=0, keepdims=True)
        jj = lax.broadcasted_iota(jnp.int32, (n, _NG), 0)
        gg = lax.broadcasted_iota(jnp.int32, (n, _NG), 1)
        w = lax.bitcast_convert_type((127 - 2 * (jj & 31)) << 23, jnp.float32)
        g_ref[...] = jnp.where((jj >> 5) == gg, w,
                               jnp.float32(0.0)).astype(jnp.bfloat16)
        gid_ref[...] = lax.broadcasted_iota(
            jnp.int32, (1, _NG), 1).astype(jnp.float32)
        acc_ref[0, 0] = 0.0
        cacc_ref[...] = jnp.zeros_like(cacc_ref)

    z = z_ref[...]
    mm = lax.dot_general(z, et_ref[...], (((1,), (0,)), ((), ())),
                         preferred_element_type=jnp.float32)
    d2 = (sumz_ref[...] + mm * -2.0) + sume_ref[...]
    dist = jnp.sqrt(jnp.maximum(d2, 0.0))
    vmin = _tree_min(dist)
    maskb = (dist == vmin).astype(jnp.bfloat16)
    pres = lax.dot_general(maskb, g_ref[...], (((1,), (0,)), ((), ())),
                           preferred_element_type=jnp.float32)
    gidf = gid_ref[...]
    firstg = _tree_min(jnp.where(pres > 0.0, gidf, jnp.float32(jnp.inf)))
    ohg = (gidf == firstg).astype(jnp.float32)
    psel = jnp.sum(pres * ohg, axis=1, keepdims=True)
    eb = (lax.bitcast_convert_type(psel, jnp.int32) >> 23) & 255
    kk = (127 - eb) >> 1
    first_i = firstg.astype(jnp.int32) * 32 + kk
    idx_ref[...] = first_i[:, 0]
    ohk = (lax.broadcasted_iota(jnp.int32, (bm, 32), 1)
           == kk).astype(jnp.bfloat16)
    cacc_ref[...] += lax.dot_general(ohg.astype(jnp.bfloat16), ohk,
                                     (((0,), (0,)), ((), ())),
                                     preferred_element_type=jnp.float32)
    acc_ref[0, 0] += jnp.sum(vmin * vmin)

    @pl.when(i == pl.num_programs(0) - 1)
    def _done():
        losssum_ref[0, 0] = acc_ref[0, 0]
        counts_ref[...] = cacc_ref[...]


def _argmin_call(z_e, sumz, emb_t):
    m, k = z_e.shape
    n = emb_t.shape[1]
    return pl.pallas_call(
        _argmin_body,
        grid=(m // _BM,),
        in_specs=[
            pl.BlockSpec((_BM, k), lambda i: (i, 0)),
            pl.BlockSpec((_BM, 1), lambda i: (i, 0)),
            pl.BlockSpec((k, n), lambda i: (0, 0)),
        ],
        out_specs=[
            pl.BlockSpec((_BM,), lambda i: (i,)),
            pl.BlockSpec(memory_space=pltpu.SMEM),
            pl.BlockSpec((_NG, 32), lambda i: (0, 0)),
        ],
        out_shape=[
            jax.ShapeDtypeStruct((m,), jnp.int32),
            jax.ShapeDtypeStruct((1, 1), jnp.float32),
            jax.ShapeDtypeStruct((_NG, 32), jnp.float32),
        ],
        scratch_shapes=[
            pltpu.SMEM((1, 1), jnp.float32),
            pltpu.VMEM((1, n), jnp.float32),
            pltpu.VMEM((n, _NG), jnp.bfloat16),
            pltpu.VMEM((1, _NG), jnp.float32),
            pltpu.VMEM((_NG, 32), jnp.float32),
        ],
        compiler_params=pltpu.CompilerParams(
            dimension_semantics=("arbitrary",)),
    )(z_e, sumz, emb_t)


_NCH = _BPW // _GCH
_STRIPE = _NUM_CODES // 16


def _sc_gather_body(emb_hbm, idx_hbm, zq_hbm, idx_v, rows_v, sem):
    wid = lax.axis_index("s") * 2 + lax.axis_index("c")
    base = wid * _BPW

    def _ldidx(j, c):
        pltpu.sync_copy(idx_hbm.at[pl.ds(base + j * _GCH, _GCH)], idx_v.at[j])
        return c
    lax.fori_loop(0, _NCH, _ldidx, 0)

    def _gather(j, c):
        pltpu.async_copy(emb_hbm.at[idx_v.at[j]], rows_v, sem).wait()
        pltpu.sync_copy(rows_v, zq_hbm.at[pl.ds(base + j * _GCH, _GCH)])
        return c
    lax.fori_loop(0, _NCH, _gather, 0)


def _sc_gather(embedding, indices):
    mesh = plsc.VectorSubcoreMesh(core_axis_name="c", subcore_axis_name="s")
    fn = functools.partial(
        pl.kernel,
        mesh=mesh,
        out_type=jax.ShapeDtypeStruct((_BATCH, _EMBED_DIM), jnp.float32),
        scratch_types=[
            pltpu.VMEM((_NCH, _GCH), jnp.int32),
            pltpu.VMEM((_GCH, _EMBED_DIM), jnp.float32),
            pltpu.SemaphoreType.DMA,
        ],
    )(_sc_gather_body)
    return fn(embedding, indices)


def _perp_body(pc_ref, out_ref):
    counts = pc_ref[...]
    p = counts / jnp.float32(_BATCH)
    ent = jnp.sum(p * jnp.log(p + 1e-10))
    out_ref[0, 0] = jnp.exp(-ent)


def _perp_call(pcounts):
    return pl.pallas_call(
        _perp_body,
        out_specs=pl.BlockSpec(memory_space=pltpu.SMEM),
        out_shape=jax.ShapeDtypeStruct((1, 1), jnp.float32),
    )(pcounts)


def kernel(z_e, embedding):
    emb_t = embedding.T
    sumz = jnp.sum(z_e * z_e, axis=1, keepdims=True)
    indices, loss_sum, counts2d = _argmin_call(z_e, sumz, emb_t)
    z_q = _sc_gather(embedding, indices)
    perp = _perp_call(counts2d.reshape(1, _NUM_CODES))
    codebook_loss = loss_sum[0, 0] / jnp.float32(_BATCH * _EMBED_DIM)
    commitment_loss = _COMMITMENT_COST * codebook_loss
    return (z_q, indices, codebook_loss, commitment_loss, perp[0, 0])
